# baseline (device time: 113584 ns/iter reference)
import jax
import jax.numpy as jnp
from jax import lax
from jax.experimental import pallas as pl
from jax.experimental.pallas import tpu as pltpu

Z = 4
M = 1024
N = 1024
CHUNK = M // Z
N_HOPS = 2 * (Z - 1)


def kernel(dy, W):
    def body(dy_ref, w_ref, out_ref, comm_ref, send_sems, recv_sems):
        my_x = lax.axis_index("x")
        my_y = lax.axis_index("y")
        my_z = lax.axis_index("z")
        right = (my_z + 1) % Z

        out_ref[...] = lax.dot_general(
            dy_ref[...],
            w_ref[...],
            dimension_numbers=(((1,), (1,)), ((), ())),
            preferred_element_type=jnp.float32,
        )

        for s in range(Z - 1):
            send_idx = (my_z - s) % Z
            recv_idx = (my_z - s - 1) % Z
            rdma = pltpu.make_async_remote_copy(
                src_ref=out_ref.at[pl.ds(send_idx * CHUNK, CHUNK), :],
                dst_ref=comm_ref.at[s],
                send_sem=send_sems.at[s],
                recv_sem=recv_sems.at[s],
                device_id=(my_x, my_y, right),
                device_id_type=pl.DeviceIdType.MESH,
            )
            rdma.start()
            rdma.wait()
            out_ref[pl.ds(recv_idx * CHUNK, CHUNK), :] += comm_ref[s]

        for s in range(Z - 1):
            h = (Z - 1) + s
            send_idx = (my_z + 1 - s) % Z
            recv_idx = (my_z - s) % Z
            rdma = pltpu.make_async_remote_copy(
                src_ref=out_ref.at[pl.ds(send_idx * CHUNK, CHUNK), :],
                dst_ref=comm_ref.at[h],
                send_sem=send_sems.at[h],
                recv_sem=recv_sems.at[h],
                device_id=(my_x, my_y, right),
                device_id_type=pl.DeviceIdType.MESH,
            )
            rdma.start()
            rdma.wait()
            out_ref[pl.ds(recv_idx * CHUNK, CHUNK), :] = comm_ref[h]

    return pl.pallas_call(
        body,
        out_shape=jax.ShapeDtypeStruct((M, N), jnp.float32),
        in_specs=[
            pl.BlockSpec(memory_space=pltpu.VMEM),
            pl.BlockSpec(memory_space=pltpu.VMEM),
        ],
        out_specs=pl.BlockSpec(memory_space=pltpu.VMEM),
        scratch_shapes=[
            pltpu.VMEM((N_HOPS, CHUNK, N), jnp.float32),
            pltpu.SemaphoreType.DMA((N_HOPS,)),
            pltpu.SemaphoreType.DMA((N_HOPS,)),
        ],
    )(dy, W)


# device time: 79897 ns/iter; 1.4216x vs baseline; 1.4216x over previous
import jax
import jax.numpy as jnp
from jax import lax
from jax.experimental import pallas as pl
from jax.experimental.pallas import tpu as pltpu

ZN = 4
P_XY = 8
M = 1024
N = 1024
BLK = M // P_XY
SUB = BLK // ZN
Z_HOPS = 2 * (ZN - 1)
CW_HOPS = P_XY // 2
CCW_HOPS = P_XY // 2 - 1


def _xy_coords(q):
    q = q % P_XY
    return (jnp.where(q < 4, 0, 1), jnp.where(q < 4, q, 7 - q))


def kernel(dy, W):
    def body(
        dy_ref,
        w_ref,
        out_ref,
        comm_z,
        zsend,
        zrecv,
        comm_cw,
        cwsend,
        cwrecv,
        comm_ccw,
        ccwsend,
        ccwrecv,
    ):
        mx = lax.axis_index("x")
        my = lax.axis_index("y")
        mz = lax.axis_index("z")
        p = jnp.where(mx == 0, my, 7 - my)
        base = p * BLK
        zright = (mz + 1) % ZN

        out_ref[pl.ds(base, BLK), :] = lax.dot_general(
            dy_ref[pl.ds(base, BLK), :],
            w_ref[...],
            dimension_numbers=(((1,), (1,)), ((), ())),
            preferred_element_type=jnp.float32,
        )

        for s in range(ZN - 1):
            send_i = (mz - s) % ZN
            recv_i = (mz - s - 1) % ZN
            rdma = pltpu.make_async_remote_copy(
                src_ref=out_ref.at[pl.ds(base + send_i * SUB, SUB), :],
                dst_ref=comm_z.at[s],
                send_sem=zsend.at[s],
                recv_sem=zrecv.at[s],
                device_id=(mx, my, zright),
                device_id_type=pl.DeviceIdType.MESH,
            )
            rdma.start()
            rdma.wait()
            out_ref[pl.ds(base + recv_i * SUB, SUB), :] += comm_z[s]
        for s in range(ZN - 1):
            h = (ZN - 1) + s
            send_i = (mz + 1 - s) % ZN
            recv_i = (mz - s) % ZN
            rdma = pltpu.make_async_remote_copy(
                src_ref=out_ref.at[pl.ds(base + send_i * SUB, SUB), :],
                dst_ref=comm_z.at[h],
                send_sem=zsend.at[h],
                recv_sem=zrecv.at[h],
                device_id=(mx, my, zright),
                device_id_type=pl.DeviceIdType.MESH,
            )
            rdma.start()
            rdma.wait()
            out_ref[pl.ds(base + recv_i * SUB, SUB), :] = comm_z[h]

        rx, ry = _xy_coords(p + 1)
        lx, ly = _xy_coords(p - 1)
        for h in range(CW_HOPS):
            src_cw = (
                out_ref.at[pl.ds(base, BLK), :] if h == 0 else comm_cw.at[h - 1]
            )
            rcw = pltpu.make_async_remote_copy(
                src_ref=src_cw,
                dst_ref=comm_cw.at[h],
                send_sem=cwsend.at[h],
                recv_sem=cwrecv.at[h],
                device_id=(rx, ry, mz),
                device_id_type=pl.DeviceIdType.MESH,
            )
            rcw.start()
            if h < CCW_HOPS:
                src_ccw = (
                    out_ref.at[pl.ds(base, BLK), :]
                    if h == 0
                    else comm_ccw.at[h - 1]
                )
                rccw = pltpu.make_async_remote_copy(
                    src_ref=src_ccw,
                    dst_ref=comm_ccw.at[h],
                    send_sem=ccwsend.at[h],
                    recv_sem=ccwrecv.at[h],
                    device_id=(lx, ly, mz),
                    device_id_type=pl.DeviceIdType.MESH,
                )
                rccw.start()
            rcw.wait()
            out_ref[pl.ds(((p - 1 - h) % P_XY) * BLK, BLK), :] = comm_cw[h]
            if h < CCW_HOPS:
                rccw.wait()
                out_ref[pl.ds(((p + 1 + h) % P_XY) * BLK, BLK), :] = comm_ccw[h]

    return pl.pallas_call(
        body,
        out_shape=jax.ShapeDtypeStruct((M, N), jnp.float32),
        in_specs=[
            pl.BlockSpec(memory_space=pltpu.VMEM),
            pl.BlockSpec(memory_space=pltpu.VMEM),
        ],
        out_specs=pl.BlockSpec(memory_space=pltpu.VMEM),
        scratch_shapes=[
            pltpu.VMEM((Z_HOPS, SUB, N), jnp.float32),
            pltpu.SemaphoreType.DMA((Z_HOPS,)),
            pltpu.SemaphoreType.DMA((Z_HOPS,)),
            pltpu.VMEM((CW_HOPS, BLK, N), jnp.float32),
            pltpu.SemaphoreType.DMA((CW_HOPS,)),
            pltpu.SemaphoreType.DMA((CW_HOPS,)),
            pltpu.VMEM((CCW_HOPS, BLK, N), jnp.float32),
            pltpu.SemaphoreType.DMA((CCW_HOPS,)),
            pltpu.SemaphoreType.DMA((CCW_HOPS,)),
        ],
    )(dy, W)


# device time: 64181 ns/iter; 1.7697x vs baseline; 1.2449x over previous
import jax
import jax.numpy as jnp
from jax import lax
from jax.experimental import pallas as pl
from jax.experimental.pallas import tpu as pltpu

ZN = 4
P_XY = 8
M = 1024
N = 1024
BLK = M // P_XY
SUB = BLK // ZN
CW_HOPS = P_XY // 2
CCW_HOPS = P_XY // 2 - 1


def _xy_coords(q):
    q = q % P_XY
    return (jnp.where(q < 4, 0, 1), jnp.where(q < 4, q, 7 - q))


def kernel(dy, W):
    def body(
        dy_ref,
        w_ref,
        out_ref,
        zrs_buf,
        zrs_send,
        zrs_recv,
        zag_buf,
        zag_send,
        zag_recv,
        comm_cw,
        cwsend,
        cwrecv,
        comm_ccw,
        ccwsend,
        ccwrecv,
    ):
        mx = lax.axis_index("x")
        my = lax.axis_index("y")
        mz = lax.axis_index("z")
        p = jnp.where(mx == 0, my, 7 - my)
        base = p * BLK
        rx, ry = _xy_coords(p + 1)
        lx, ly = _xy_coords(p - 1)

        bsem = pltpu.get_barrier_semaphore()
        for k in range(1, ZN):
            pl.semaphore_signal(
                bsem,
                inc=1,
                device_id=(mx, my, (mz + k) % ZN),
                device_id_type=pl.DeviceIdType.MESH,
            )
        for tgt in ((rx, ry, mz), (lx, ly, mz)):
            pl.semaphore_signal(
                bsem, inc=1, device_id=tgt, device_id_type=pl.DeviceIdType.MESH
            )
        pl.semaphore_wait(bsem, ZN - 1 + 2)

        out_ref[pl.ds(base, BLK), :] = lax.dot_general(
            dy_ref[pl.ds(base, BLK), :],
            w_ref[...],
            dimension_numbers=(((1,), (1,)), ((), ())),
            preferred_element_type=jnp.float32,
        )

        zrs = []
        for k in range(1, ZN):
            tz = (mz + k) % ZN
            slot = (mz - tz - 1) % ZN
            r = pltpu.make_async_remote_copy(
                src_ref=out_ref.at[pl.ds(base + tz * SUB, SUB), :],
                dst_ref=zrs_buf.at[slot],
                send_sem=zrs_send.at[k - 1],
                recv_sem=zrs_recv.at[slot],
                device_id=(mx, my, tz),
                device_id_type=pl.DeviceIdType.MESH,
            )
            r.start()
            zrs.append(r)
        for r in zrs:
            r.wait_recv()
        out_ref[pl.ds(base + mz * SUB, SUB), :] += (
            zrs_buf[0] + zrs_buf[1] + zrs_buf[2]
        )

        zag = []
        for k in range(1, ZN):
            tz = (mz + k) % ZN
            slot = (mz - tz - 1) % ZN
            r = pltpu.make_async_remote_copy(
                src_ref=out_ref.at[pl.ds(base + mz * SUB, SUB), :],
                dst_ref=zag_buf.at[slot],
                send_sem=zag_send.at[k - 1],
                recv_sem=zag_recv.at[slot],
                device_id=(mx, my, tz),
                device_id_type=pl.DeviceIdType.MESH,
            )
            r.start()
            zag.append(r)
        for r in zrs:
            r.wait_send()
        for r in zag:
            r.wait_recv()
        for k in range(1, ZN):
            tz = (mz + k) % ZN
            out_ref[pl.ds(base + tz * SUB, SUB), :] = zag_buf[k - 1]

        def mk_cw(h):
            src = out_ref.at[pl.ds(base, BLK), :] if h == 0 else comm_cw.at[h - 1]
            return pltpu.make_async_remote_copy(
                src_ref=src,
                dst_ref=comm_cw.at[h],
                send_sem=cwsend.at[h],
                recv_sem=cwrecv.at[h],
                device_id=(rx, ry, mz),
                device_id_type=pl.DeviceIdType.MESH,
            )

        def mk_ccw(h):
            src = (
                out_ref.at[pl.ds(base, BLK), :] if h == 0 else comm_ccw.at[h - 1]
            )
            return pltpu.make_async_remote_copy(
                src_ref=src,
                dst_ref=comm_ccw.at[h],
                send_sem=ccwsend.at[h],
                recv_sem=ccwrecv.at[h],
                device_id=(lx, ly, mz),
                device_id_type=pl.DeviceIdType.MESH,
            )

        cw = [mk_cw(0)]
        ccw = [mk_ccw(0)]
        cw[0].start()
        ccw[0].start()
        for h in range(CW_HOPS):
            cw[h].wait_recv()
            if h + 1 < CW_HOPS:
                cw.append(mk_cw(h + 1))
                cw[h + 1].start()
            if h < CCW_HOPS:
                ccw[h].wait_recv()
                if h + 1 < CCW_HOPS:
                    ccw.append(mk_ccw(h + 1))
                    ccw[h + 1].start()
            out_ref[pl.ds(((p - 1 - h) % P_XY) * BLK, BLK), :] = comm_cw[h]
            if h < CCW_HOPS:
                out_ref[pl.ds(((p + 1 + h) % P_XY) * BLK, BLK), :] = comm_ccw[h]
        for r in zag:
            r.wait_send()
        for r in cw:
            r.wait_send()
        for r in ccw:
            r.wait_send()

    return pl.pallas_call(
        body,
        out_shape=jax.ShapeDtypeStruct((M, N), jnp.float32),
        in_specs=[
            pl.BlockSpec(memory_space=pltpu.VMEM),
            pl.BlockSpec(memory_space=pltpu.VMEM),
        ],
        out_specs=pl.BlockSpec(memory_space=pltpu.VMEM),
        scratch_shapes=[
            pltpu.VMEM((ZN - 1, SUB, N), jnp.float32),
            pltpu.SemaphoreType.DMA((ZN - 1,)),
            pltpu.SemaphoreType.DMA((ZN - 1,)),
            pltpu.VMEM((ZN - 1, SUB, N), jnp.float32),
            pltpu.SemaphoreType.DMA((ZN - 1,)),
            pltpu.SemaphoreType.DMA((ZN - 1,)),
            pltpu.VMEM((CW_HOPS, BLK, N), jnp.float32),
            pltpu.SemaphoreType.DMA((CW_HOPS,)),
            pltpu.SemaphoreType.DMA((CW_HOPS,)),
            pltpu.VMEM((CCW_HOPS, BLK, N), jnp.float32),
            pltpu.SemaphoreType.DMA((CCW_HOPS,)),
            pltpu.SemaphoreType.DMA((CCW_HOPS,)),
        ],
        compiler_params=pltpu.CompilerParams(collective_id=0),
    )(dy, W)


# device time: 59378 ns/iter; 1.9129x vs baseline; 1.0809x over previous
import jax
import jax.numpy as jnp
from jax import lax
from jax.experimental import pallas as pl
from jax.experimental.pallas import tpu as pltpu

ZN = 4
P_XY = 8
M = 1024
N = 1024
BLK = M // P_XY
SUB = BLK // ZN
HOPS = 4


def _xy_coords(q):
    q = q % P_XY
    return (jnp.where(q < 4, 0, 1), jnp.where(q < 4, q, 7 - q))


def kernel(dy, W):
    def body(
        dy_ref,
        w_ref,
        out_ref,
        zrs_buf,
        zrs_send,
        zrs_recv,
        zag_buf,
        zag_send,
        zag_recv,
        comm_cw,
        cw0send,
        cw0recv,
        cwsend,
        cwrecv,
        comm_ccw,
        ccw0send,
        ccw0recv,
        ccwsend,
        ccwrecv,
    ):
        mx = lax.axis_index("x")
        my = lax.axis_index("y")
        mz = lax.axis_index("z")
        p = jnp.where(mx == 0, my, 7 - my)
        base = p * BLK
        rx, ry = _xy_coords(p + 1)
        lx, ly = _xy_coords(p - 1)

        bsem = pltpu.get_barrier_semaphore()
        for k in range(1, ZN):
            pl.semaphore_signal(
                bsem,
                inc=1,
                device_id=(mx, my, (mz + k) % ZN),
                device_id_type=pl.DeviceIdType.MESH,
            )
        for tgt in ((rx, ry, mz), (lx, ly, mz)):
            pl.semaphore_signal(
                bsem, inc=1, device_id=tgt, device_id_type=pl.DeviceIdType.MESH
            )
        pl.semaphore_wait(bsem, ZN - 1 + 2)

        out_ref[pl.ds(base, BLK), :] = lax.dot_general(
            dy_ref[pl.ds(base, BLK), :],
            w_ref[...],
            dimension_numbers=(((1,), (1,)), ((), ())),
            preferred_element_type=jnp.float32,
        )

        zrs = []
        for k in range(1, ZN):
            tz = (mz + k) % ZN
            slot = (mz - tz - 1) % ZN
            r = pltpu.make_async_remote_copy(
                src_ref=out_ref.at[pl.ds(base + tz * SUB, SUB), :],
                dst_ref=zrs_buf.at[slot],
                send_sem=zrs_send.at[k - 1],
                recv_sem=zrs_recv.at[slot],
                device_id=(mx, my, tz),
                device_id_type=pl.DeviceIdType.MESH,
            )
            r.start()
            zrs.append(r)
        for r in zrs:
            r.wait_recv()
        out_ref[pl.ds(base + mz * SUB, SUB), :] += (
            zrs_buf[0] + zrs_buf[1] + zrs_buf[2]
        )

        def mk_hop0(j, src, dst_buf, sends, recvs, dev):
            return pltpu.make_async_remote_copy(
                src_ref=src,
                dst_ref=dst_buf.at[0, pl.ds(j * SUB, SUB), :],
                send_sem=sends.at[j],
                recv_sem=recvs.at[j],
                device_id=dev,
                device_id_type=pl.DeviceIdType.MESH,
            )

        my_slice = out_ref.at[pl.ds(base + mz * SUB, SUB), :]
        cw0 = [mk_hop0(0, my_slice, comm_cw, cw0send, cw0recv, (rx, ry, mz))]
        ccw0 = [mk_hop0(0, my_slice, comm_ccw, ccw0send, ccw0recv, (lx, ly, mz))]
        cw0[0].start()
        ccw0[0].start()

        zag = []
        for k in range(1, ZN):
            tz = (mz + k) % ZN
            slot = (mz - tz - 1) % ZN
            r = pltpu.make_async_remote_copy(
                src_ref=out_ref.at[pl.ds(base + mz * SUB, SUB), :],
                dst_ref=zag_buf.at[slot],
                send_sem=zag_send.at[k - 1],
                recv_sem=zag_recv.at[slot],
                device_id=(mx, my, tz),
                device_id_type=pl.DeviceIdType.MESH,
            )
            r.start()
            zag.append(r)
        for r in zrs:
            r.wait_send()
        for k in range(1, ZN):
            rk = pltpu.make_async_remote_copy(
                src_ref=out_ref.at[pl.ds(base + mz * SUB, SUB), :],
                dst_ref=zag_buf.at[k - 1],
                send_sem=zag_send.at[k - 1],
                recv_sem=zag_recv.at[k - 1],
                device_id=(mx, my, mz),
                device_id_type=pl.DeviceIdType.MESH,
            )
            rk.wait_recv()
            src = zag_buf.at[k - 1]
            cw0.append(mk_hop0(k, src, comm_cw, cw0send, cw0recv, (rx, ry, mz)))
            ccw0.append(
                mk_hop0(k, src, comm_ccw, ccw0send, ccw0recv, (lx, ly, mz))
            )
            cw0[k].start()
            ccw0[k].start()
            tz = (mz + k) % ZN
            out_ref[pl.ds(base + tz * SUB, SUB), :] = zag_buf[k - 1]

        def store_block(buf, h, blk_pos, slots):
            b = (blk_pos % P_XY) * BLK
            for j in slots:
                out_ref[pl.ds(b + ((mz + j) % ZN) * SUB, SUB), :] = buf[
                    h, pl.ds(j * SUB, SUB), :
                ]

        def mk_fwd(h, buf, sends, recvs, dev, half=None):
            if half is None:
                src = buf.at[h - 1]
                dst = buf.at[h]
            else:
                lo = 0 if half == "lo" else 2 * SUB
                src = buf.at[h - 1, pl.ds(lo, 2 * SUB), :]
                dst = buf.at[h, pl.ds(lo, 2 * SUB), :]
            return pltpu.make_async_remote_copy(
                src_ref=src,
                dst_ref=dst,
                send_sem=sends.at[h - 1],
                recv_sem=recvs.at[h - 1],
                device_id=dev,
                device_id_type=pl.DeviceIdType.MESH,
            )

        cw = {}
        ccw = {}
        for r in cw0:
            r.wait_recv()
        cw[1] = mk_fwd(1, comm_cw, cwsend, cwrecv, (rx, ry, mz))
        cw[1].start()
        for r in ccw0:
            r.wait_recv()
        ccw[1] = mk_fwd(1, comm_ccw, ccwsend, ccwrecv, (lx, ly, mz))
        ccw[1].start()
        store_block(comm_cw, 0, p - 1, range(ZN))
        store_block(comm_ccw, 0, p + 1, range(ZN))
        for h in (1, 2):
            nxt_half = ("lo", "hi") if h == 2 else (None, None)
            cw[h].wait_recv()
            cw[h + 1] = mk_fwd(
                h + 1, comm_cw, cwsend, cwrecv, (rx, ry, mz), half=nxt_half[0]
            )
            cw[h + 1].start()
            ccw[h].wait_recv()
            ccw[h + 1] = mk_fwd(
                h + 1, comm_ccw, ccwsend, ccwrecv, (lx, ly, mz), half=nxt_half[1]
            )
            ccw[h + 1].start()
            store_block(comm_cw, h, p - 1 - h, range(ZN))
            store_block(comm_ccw, h, p + 1 + h, range(ZN))
        cw[3].wait_recv()
        store_block(comm_cw, 3, p - 4, (0, 1))
        ccw[3].wait_recv()
        store_block(comm_ccw, 3, p - 4, (2, 3))

        for r in zag:
            r.wait_send()
        for r in cw0 + ccw0:
            r.wait_send()
        for h in (1, 2, 3):
            cw[h].wait_send()
            ccw[h].wait_send()

    return pl.pallas_call(
        body,
        out_shape=jax.ShapeDtypeStruct((M, N), jnp.float32),
        in_specs=[
            pl.BlockSpec(memory_space=pltpu.VMEM),
            pl.BlockSpec(memory_space=pltpu.VMEM),
        ],
        out_specs=pl.BlockSpec(memory_space=pltpu.VMEM),
        scratch_shapes=[
            pltpu.VMEM((ZN - 1, SUB, N), jnp.float32),
            pltpu.SemaphoreType.DMA((ZN - 1,)),
            pltpu.SemaphoreType.DMA((ZN - 1,)),
            pltpu.VMEM((ZN - 1, SUB, N), jnp.float32),
            pltpu.SemaphoreType.DMA((ZN - 1,)),
            pltpu.SemaphoreType.DMA((ZN - 1,)),
            pltpu.VMEM((HOPS, BLK, N), jnp.float32),
            pltpu.SemaphoreType.DMA((ZN,)),
            pltpu.SemaphoreType.DMA((ZN,)),
            pltpu.SemaphoreType.DMA((HOPS - 1,)),
            pltpu.SemaphoreType.DMA((HOPS - 1,)),
            pltpu.VMEM((HOPS, BLK, N), jnp.float32),
            pltpu.SemaphoreType.DMA((ZN,)),
            pltpu.SemaphoreType.DMA((ZN,)),
            pltpu.SemaphoreType.DMA((HOPS - 1,)),
            pltpu.SemaphoreType.DMA((HOPS - 1,)),
        ],
        compiler_params=pltpu.CompilerParams(collective_id=0),
    )(dy, W)
